# baseline (speedup 1.0000x reference)
import functools

import jax
import jax.numpy as jnp
from jax import lax
from jax.experimental import pallas as pl
from jax.experimental.pallas import tpu as pltpu
from jax.experimental.pallas import tpu_sc as plsc

BATCH = 16384
DIM = 64
NUM_TBL = 1000000
PHYS_COLS = 1000064
NUM_CORES = 2
NUM_SUBCORES = 16
NUM_WORKERS = NUM_CORES * NUM_SUBCORES
R_COLS = 31360
BLK = 512
NBLK = R_COLS // BLK + 1
NRING = 3
LAST_SO = PHYS_COLS - BLK
HITCAP = 2048
DUMP_ROW = BATCH
STAGE = 16
SENTINEL = 2**30


def _emb_body(idx_hbm, wt_hbm, out_hbm, idx_v, blkbuf, hits, tmp,
              rowstage, bidx, sems):
    wid = lax.axis_index("s") * NUM_CORES + lax.axis_index("c")
    w_lo = wid * R_COLS
    w_hi = jnp.minimum(w_lo + R_COLS, NUM_TBL)
    lane = lax.iota(jnp.int32, 16)

    def fetch(g, buf):
        s = w_lo + g * BLK

        @pl.when(s < NUM_TBL)
        def _():
            so = pl.multiple_of(jnp.minimum(s, LAST_SO), 128)
            for a in range(DIM // 8):
                pltpu.async_copy(
                    wt_hbm.at[pl.ds(8 * a, 8), pl.ds(so, BLK)],
                    blkbuf.at[buf, pl.ds(8 * a, 8)],
                    sems.at[buf],
                )

    def wait_fetch(g, buf):
        s = w_lo + g * BLK

        @pl.when(s < NUM_TBL)
        def _():
            pltpu.make_async_copy(
                wt_hbm.at[:, pl.ds(0, BLK)], blkbuf.at[buf], sems.at[buf]
            ).wait()

    for g in range(NRING):
        fetch(g, g)

    pltpu.sync_copy(idx_hbm, idx_v)

    def scan_body(v, n):
        hv = idx_v[pl.ds(v * 16, 16)]
        m = jnp.logical_and(hv >= w_lo, hv < w_hi)
        cnt = plsc.all_reduce_population_count(m)[0]

        @pl.when(cnt > 0)
        def _():
            packed = (hv - w_lo) | ((v * 16 + lane) << 15)
            plsc.store_compressed(hits.at[pl.ds(n, 16)], packed, mask=m)

        return jnp.minimum(n + cnt, HITCAP - 16)

    n_hits = lax.fori_loop(0, BATCH // 16, scan_body, 0)
    hits[pl.ds(n_hits, 16)] = jnp.full((16,), SENTINEL, jnp.int32)
    n_hvregs = (n_hits + 15) >> 4

    bidx[...] = jnp.full((16,), jnp.int32(DUMP_ROW))

    def scalar_at(ref, k):
        return plsc.load_gather(ref, [jnp.full((16,), k, jnp.int32)])[0]

    dvecs = [16 * q + lane for q in range(4)]

    def flush():
        pltpu.sync_copy(rowstage, out_hbm.at[bidx])
        bidx[...] = jnp.full((16,), jnp.int32(DUMP_ROW))

    def block_body(g, slot):
        buf = lax.rem(g, NRING)
        wait_fetch(g, buf)
        s = w_lo + g * BLK
        so = jnp.minimum(s, LAST_SO)
        rel_so = so - w_lo
        bufv = jnp.full((16,), buf, jnp.int32)

        def match_body(v, slot):
            hp = hits[pl.ds(v * 16, 16)]
            m = (jnp.bitwise_and(hp, 0x7FFF) >> 9) == g
            cnt = plsc.all_reduce_population_count(m)[0]
            plsc.store_compressed(tmp.at[pl.ds(0, 16)], hp, mask=m)

            def ext_cond(carry):
                k, _ = carry
                return k < cnt

            def ext_body(carry):
                k, slot = carry
                hp_k = scalar_at(tmp, k)
                l = jnp.bitwise_and(hp_k, 0x7FFF) - rel_so
                b = hp_k >> 15
                lv = jnp.full((16,), l, jnp.int32)
                for q in range(4):
                    g16 = plsc.load_gather(blkbuf, [bufv, dvecs[q], lv])
                    rowstage[slot, pl.ds(16 * q, 16)] = g16
                plsc.store_scatter(bidx, [jnp.full((16,), slot, jnp.int32)],
                                   jnp.full((16,), b, jnp.int32),
                                   mask=lane == 0)
                slot = slot + 1

                @pl.when(slot == STAGE)
                def _():
                    flush()

                return k + 1, lax.rem(slot, STAGE)

            _, slot = lax.while_loop(ext_cond, ext_body, (0, slot))
            return slot

        slot = lax.fori_loop(0, n_hvregs, match_body, slot)

        @pl.when(g + NRING < NBLK)
        def _():
            fetch(g + NRING, buf)

        return slot

    slot = lax.fori_loop(0, NBLK, block_body, 0)

    @pl.when(slot > 0)
    def _():
        flush()


def _tr_body(scr_hbm, outt_hbm, stage, outv):
    wid = lax.axis_index("s") * NUM_CORES + lax.axis_index("c")
    base = wid * (BATCH // NUM_WORKERS)
    lane = lax.iota(jnp.int32, 16)
    pltpu.sync_copy(scr_hbm.at[pl.ds(base, BATCH // NUM_WORKERS), :], stage)

    def jj_body(jj, _):
        jv = jj * 16 + lane

        def d_body(d, _):
            g16 = plsc.load_gather(stage, [jv, jnp.full((16,), d, jnp.int32)])
            outv[d, pl.ds(jj * 16, 16)] = g16
            return 0

        return lax.fori_loop(0, DIM, d_body, 0)

    lax.fori_loop(0, (BATCH // NUM_WORKERS) // 16, jj_body, 0)
    pltpu.sync_copy(outv, outt_hbm.at[:, pl.ds(base, BATCH // NUM_WORKERS)])


@jax.jit
def _embed(indices, weight):
    mesh = plsc.VectorSubcoreMesh(core_axis_name="c", subcore_axis_name="s")
    params = pltpu.CompilerParams(
        use_tc_tiling_on_sc=True, needs_layout_passes=False
    )
    scratch = pl.kernel(
        _emb_body,
        mesh=mesh,
        out_type=jax.ShapeDtypeStruct((BATCH + 128, 128), jnp.float32),
        scratch_types=[
            pltpu.VMEM((BATCH,), jnp.int32),
            pltpu.VMEM((NRING, DIM, BLK), jnp.float32),
            pltpu.VMEM((HITCAP,), jnp.int32),
            pltpu.VMEM((16,), jnp.int32),
            pltpu.VMEM((STAGE, 128), jnp.float32),
            pltpu.VMEM((16,), jnp.int32),
            pltpu.SemaphoreType.DMA((NRING,)),
        ],
        compiler_params=params,
    )(indices, weight.T)
    out_t = pl.kernel(
        _tr_body,
        mesh=mesh,
        out_type=jax.ShapeDtypeStruct((DIM, BATCH), jnp.float32),
        scratch_types=[
            pltpu.VMEM((BATCH // NUM_WORKERS, 128), jnp.float32),
            pltpu.VMEM((DIM, BATCH // NUM_WORKERS), jnp.float32),
        ],
        compiler_params=params,
    )(scratch)
    return out_t.T


def kernel(indices, weight):
    return _embed(indices.astype(jnp.int32), weight)


# speedup vs baseline: 1.0997x; 1.0997x over previous
import functools

import jax
import jax.numpy as jnp
from jax import lax
from jax.experimental import pallas as pl
from jax.experimental.pallas import tpu as pltpu
from jax.experimental.pallas import tpu_sc as plsc

BATCH = 16384
DIM = 64
NUM_TBL = 1000000
PHYS_COLS = 1000064
NUM_CORES = 2
NUM_SUBCORES = 16
NUM_WORKERS = NUM_CORES * NUM_SUBCORES
R_COLS = 31360
BLK = 512
NBLK = R_COLS // BLK + 1
NRING = 3
LAST_SO = PHYS_COLS - BLK
HITCAP = 2048
DUMP_ROW = BATCH
STAGE = 16
SENTINEL = 2**30


def _emb_body(idx_hbm, wt_hbm, out_hbm, idx_v, blkbuf, hits, tmp,
              rowstage, bidx, sems, scat_sems, nf_ref):
    wid = lax.axis_index("s") * NUM_CORES + lax.axis_index("c")
    w_lo = wid * R_COLS
    w_hi = jnp.minimum(w_lo + R_COLS, NUM_TBL)
    lane = lax.iota(jnp.int32, 16)

    def fetch(g, buf):
        s = w_lo + g * BLK

        @pl.when(s < NUM_TBL)
        def _():
            so = pl.multiple_of(jnp.minimum(s, LAST_SO), 128)
            for a in range(DIM // 8):
                pltpu.async_copy(
                    wt_hbm.at[pl.ds(8 * a, 8), pl.ds(so, BLK)],
                    blkbuf.at[buf, pl.ds(8 * a, 8)],
                    sems.at[buf],
                )

    def wait_fetch(g, buf):
        s = w_lo + g * BLK

        @pl.when(s < NUM_TBL)
        def _():
            pltpu.make_async_copy(
                wt_hbm.at[:, pl.ds(0, BLK)], blkbuf.at[buf], sems.at[buf]
            ).wait()

    for g in range(NRING):
        fetch(g, g)

    pltpu.sync_copy(idx_hbm, idx_v)

    def scan_body(v, n):
        hv = idx_v[pl.ds(v * 16, 16)]
        m = jnp.logical_and(hv >= w_lo, hv < w_hi)
        cnt = plsc.all_reduce_population_count(m)[0]

        @pl.when(cnt > 0)
        def _():
            packed = (hv - w_lo) | ((v * 16 + lane) << 15)
            plsc.store_compressed(hits.at[pl.ds(n, 16)], packed, mask=m)

        return jnp.minimum(n + cnt, HITCAP - 16)

    n_hits = lax.fori_loop(0, BATCH // 16, scan_body, 0)
    hits[pl.ds(n_hits, 16)] = jnp.full((16,), SENTINEL, jnp.int32)
    n_hvregs = (n_hits + 15) >> 4

    bidx[0] = jnp.full((16,), jnp.int32(DUMP_ROW))
    bidx[1] = jnp.full((16,), jnp.int32(DUMP_ROW))
    nf_ref[0] = 0

    def scalar_at(ref, k):
        return plsc.load_gather(ref, [jnp.full((16,), k, jnp.int32)])[0]

    dvecs = [16 * q + lane for q in range(4)]

    def wait_scat(p):
        pltpu.make_async_copy(
            rowstage.at[p], out_hbm.at[bidx.at[p]], scat_sems.at[p]
        ).wait()

    def flush():
        nf = nf_ref[0]
        p = lax.rem(nf, 2)
        pltpu.async_copy(rowstage.at[p], out_hbm.at[bidx.at[p]],
                         scat_sems.at[p])
        nf_ref[0] = nf + 1

    def block_body(g, slot):
        buf = lax.rem(g, NRING)
        wait_fetch(g, buf)
        s = w_lo + g * BLK
        so = jnp.minimum(s, LAST_SO)
        rel_so = so - w_lo
        bufv = jnp.full((16,), buf, jnp.int32)

        def match_body(v, slot):
            hp = hits[pl.ds(v * 16, 16)]
            m = (jnp.bitwise_and(hp, 0x7FFF) >> 9) == g
            cnt = plsc.all_reduce_population_count(m)[0]
            plsc.store_compressed(tmp.at[pl.ds(0, 16)], hp, mask=m)

            def ext_cond(carry):
                k, _ = carry
                return k < cnt

            def ext_body(carry):
                k, slot = carry
                nf = nf_ref[0]
                p = lax.rem(nf, 2)

                @pl.when(slot == 0)
                def _():
                    @pl.when(nf >= 2)
                    def _():
                        wait_scat(p)

                    bidx[p] = jnp.full((16,), jnp.int32(DUMP_ROW))

                hp_k = scalar_at(tmp, k)
                l = jnp.bitwise_and(hp_k, 0x7FFF) - rel_so
                b = hp_k >> 15
                lv = jnp.full((16,), l, jnp.int32)
                for q in range(4):
                    g16 = plsc.load_gather(blkbuf, [bufv, dvecs[q], lv])
                    rowstage[p, slot, pl.ds(16 * q, 16)] = g16
                plsc.store_scatter(bidx.at[p],
                                   [jnp.full((16,), slot, jnp.int32)],
                                   jnp.full((16,), b, jnp.int32),
                                   mask=lane == 0)
                slot = slot + 1

                @pl.when(slot == STAGE)
                def _():
                    flush()

                return k + 1, lax.rem(slot, STAGE)

            _, slot = lax.while_loop(ext_cond, ext_body, (0, slot))
            return slot

        slot = lax.fori_loop(0, n_hvregs, match_body, slot)

        @pl.when(g + NRING < NBLK)
        def _():
            fetch(g + NRING, buf)

        return slot

    slot = lax.fori_loop(0, NBLK, block_body, 0)

    @pl.when(slot > 0)
    def _():
        flush()

    nf_end = nf_ref[0]

    @pl.when(nf_end >= 1)
    def _():
        wait_scat(lax.rem(nf_end - 1, 2))

    @pl.when(nf_end >= 2)
    def _():
        wait_scat(lax.rem(nf_end - 2, 2))


@jax.jit
def _embed(indices, weight):
    mesh = plsc.VectorSubcoreMesh(core_axis_name="c", subcore_axis_name="s")
    out_pad = pl.kernel(
        _emb_body,
        mesh=mesh,
        out_type=jax.ShapeDtypeStruct((BATCH + 128, 128), jnp.float32),
        scratch_types=[
            pltpu.VMEM((BATCH,), jnp.int32),
            pltpu.VMEM((NRING, DIM, BLK), jnp.float32),
            pltpu.VMEM((HITCAP,), jnp.int32),
            pltpu.VMEM((16,), jnp.int32),
            pltpu.VMEM((2, STAGE, 128), jnp.float32),
            pltpu.VMEM((2, 16), jnp.int32),
            pltpu.SemaphoreType.DMA((NRING,)),
            pltpu.SemaphoreType.DMA((2,)),
            pltpu.SMEM((1,), jnp.int32),
        ],
        compiler_params=pltpu.CompilerParams(
            use_tc_tiling_on_sc=True, needs_layout_passes=False
        ),
    )(indices, weight.T)
    return out_pad[:BATCH, :DIM]


def kernel(indices, weight):
    return _embed(indices.astype(jnp.int32), weight)
